# Initial kernel scaffold; baseline (speedup 1.0000x reference)
import jax
import jax.numpy as jnp
from jax.experimental import pallas as pl


def kernel(src_ids, edge_index1, etype1, norm1, edge_index2, etype2, norm2, emb, V1, comp1, V2, comp2):
    raise NotImplementedError("write your pallas kernel here")


# speedup vs baseline: 3.9703x; 3.9703x over previous
import functools

import jax
import jax.numpy as jnp
from jax import lax
from jax.experimental import pallas as pl
from jax.experimental.pallas import tpu as pltpu
from jax.experimental.pallas import tpu_sc as plsc

N_NODES = 10000
E_EDGES = 160000
H_DIM = 128
OUT_DIM = 128
HALF = OUT_DIM // 2
N_REL = 8
N_BASES = 2

NC = 2
NS = 16
LANES = 16

f32 = jnp.float32
i32 = jnp.int32

EC = 128
NCHUNK = 80
E_PAD = NS * NCHUNK * EC

XC = 80
XCHUNKS = 4
XROWS_W = XC * XCHUNKS
N_PAD = NC * NS * XROWS_W

N_ACC = 10240
ROWS_PER_SUB = N_ACC // NS
NZROWS = 128

_mesh = plsc.VectorSubcoreMesh(
    core_axis_name="c", subcore_axis_name="s", num_cores=NC, num_subcores=NS)


@functools.partial(
    pl.kernel,
    out_type=jax.ShapeDtypeStruct((N_PAD, H_DIM), f32),
    mesh=_mesh,
    scratch_types=[
        pltpu.VMEM((XCHUNKS, XC), i32),
        pltpu.VMEM((XROWS_W, H_DIM), f32),
        pltpu.SemaphoreType.DMA,
    ],
)
def _gather_rows(ids_hbm, table_hbm, x_hbm, idx_v, rows_v, sem):
    c = lax.axis_index("c")
    s = lax.axis_index("s")
    wid = c * NS + s
    pltpu.sync_copy(ids_hbm.at[wid], idx_v)
    for i in range(XCHUNKS):
        pltpu.async_copy(table_hbm.at[idx_v.at[i]],
                         rows_v.at[pl.ds(i * XC, XC)], sem)
    for i in range(XCHUNKS):
        pltpu.make_async_copy(table_hbm.at[idx_v.at[i]],
                              rows_v.at[pl.ds(i * XC, XC)], sem).wait()
    pltpu.sync_copy(rows_v, x_hbm.at[pl.ds(wid * XROWS_W, XROWS_W)])


@functools.partial(
    pl.kernel,
    out_type=jax.ShapeDtypeStruct((NC, N_ACC, HALF), f32),
    mesh=_mesh,
    compiler_params=pltpu.CompilerParams(use_tc_tiling_on_sc=False),
    scratch_types=[
        pltpu.VMEM((NCHUNK, EC), i32),
        pltpu.VMEM((NCHUNK, EC), i32),
        pltpu.VMEM((EC, LANES), f32),
        pltpu.VMEM((EC, LANES), f32),
        pltpu.VMEM((EC, HALF), f32),
        pltpu.VMEM((EC, HALF), f32),
        pltpu.VMEM((NZROWS, HALF), f32),
        pltpu.VMEM_SHARED((N_ACC, HALF), f32),
        pltpu.SemaphoreType.DMA,
        pltpu.SemaphoreType.DMA,
        pltpu.SemaphoreType.DMA,
        pltpu.SemaphoreType.DMA,
    ],
)
def _edge_pass(y_hbm, gidx_hbm, dst_hbm, norm_hbm, zeros_hbm, out_hbm,
               idx_v, dst_v, nb0, nb1, rows0, rows1, zero_v, acc,
               sem0, sem1, nsem0, nsem1):
    c = lax.axis_index("c")
    s = lax.axis_index("s")

    pltpu.sync_copy(zeros_hbm, zero_v)
    for z in range(ROWS_PER_SUB // NZROWS):
        pltpu.sync_copy(zero_v,
                        acc.at[pl.ds(s * ROWS_PER_SUB + z * NZROWS, NZROWS)])

    pltpu.sync_copy(gidx_hbm.at[s], idx_v)
    pltpu.sync_copy(dst_hbm.at[s], dst_v)
    plsc.subcore_barrier()

    yc = y_hbm.at[c]

    pltpu.async_copy(yc.at[idx_v.at[0]], rows0, sem0)
    pltpu.async_copy(norm_hbm.at[s, 0], nb0, nsem0)
    pltpu.async_copy(yc.at[idx_v.at[1]], rows1, sem1)
    pltpu.async_copy(norm_hbm.at[s, 1], nb1, nsem1)

    def process(chunk, rows_ref, nb_ref, sem, nsem):
        pltpu.make_async_copy(yc.at[idx_v.at[chunk]], rows_ref, sem).wait()
        pltpu.make_async_copy(norm_hbm.at[s, chunk], nb_ref, nsem).wait()

        def scale_one(k, carry):
            nrm = nb_ref[k]
            for cb in range(HALF // LANES):
                sl = rows_ref[k, pl.ds(cb * LANES, LANES)]
                rows_ref[k, pl.ds(cb * LANES, LANES)] = sl * nrm
            return carry

        lax.fori_loop(0, EC, scale_one, 0)
        pltpu.sync_copy(rows_ref, acc.at[dst_v.at[chunk]], add=True)

        @pl.when(chunk + 2 < NCHUNK)
        def _():
            pltpu.async_copy(yc.at[idx_v.at[chunk + 2]], rows_ref, sem)
            pltpu.async_copy(norm_hbm.at[s, chunk + 2], nb_ref, nsem)

    def chunk_body(j):
        process(j, rows0, nb0, sem0, nsem0)
        process(j + 1, rows1, nb1, sem1, nsem1)

    pl.loop(0, NCHUNK, step=2)(chunk_body)
    plsc.subcore_barrier()

    for z in range(ROWS_PER_SUB // NZROWS):
        r0 = s * ROWS_PER_SUB + z * NZROWS
        pltpu.sync_copy(acc.at[pl.ds(r0, NZROWS)],
                        out_hbm.at[c, pl.ds(r0, NZROWS)])


BLK = 400
NBLK = N_NODES // BLK


def _split_write(y_ref, r, res):
    y_ref[0, r] = res[:, :HALF]
    y_ref[1, r] = res[:, HALF:]


def _project_body(x_ref, v_ref, comp_ref, y_ref):
    x = x_ref[...]
    xv0 = jnp.dot(x, v_ref[0], preferred_element_type=f32)
    xv1 = jnp.dot(x, v_ref[1], preferred_element_type=f32)
    for r in range(N_REL):
        _split_write(y_ref, r, comp_ref[r, 0] * xv0 + comp_ref[r, 1] * xv1)


_project = pl.pallas_call(
    _project_body,
    grid=(NBLK,),
    in_specs=[
        pl.BlockSpec((BLK, H_DIM), lambda i: (i, 0)),
        pl.BlockSpec((N_BASES, H_DIM, OUT_DIM), lambda i: (0, 0, 0)),
        pl.BlockSpec(memory_space=pltpu.SMEM),
    ],
    out_specs=pl.BlockSpec((NC, N_REL, BLK, HALF), lambda i: (0, 0, i, 0)),
    out_shape=jax.ShapeDtypeStruct((NC, N_REL, N_NODES, HALF), f32),
)


def _combine_project_body(p_ref, v_ref, comp_ref, y_ref):
    h = jnp.concatenate([p_ref[0, 0], p_ref[1, 0]], axis=-1)
    h = jnp.maximum(h, 0.0)
    xv0 = jnp.dot(h, v_ref[0], preferred_element_type=f32)
    xv1 = jnp.dot(h, v_ref[1], preferred_element_type=f32)
    for r in range(N_REL):
        _split_write(y_ref, r, comp_ref[r, 0] * xv0 + comp_ref[r, 1] * xv1)


_combine_project = pl.pallas_call(
    _combine_project_body,
    grid=(NBLK,),
    in_specs=[
        pl.BlockSpec((NC, 1, BLK, HALF), lambda i: (0, 0, i, 0)),
        pl.BlockSpec((N_BASES, H_DIM, OUT_DIM), lambda i: (0, 0, 0)),
        pl.BlockSpec(memory_space=pltpu.SMEM),
    ],
    out_specs=pl.BlockSpec((NC, N_REL, BLK, HALF), lambda i: (0, 0, i, 0)),
    out_shape=jax.ShapeDtypeStruct((NC, N_REL, N_NODES, HALF), f32),
)


def _final_body(p_ref, o_ref):
    o_ref[...] = jnp.concatenate([p_ref[0, 0], p_ref[1, 0]], axis=-1)


_final = pl.pallas_call(
    _final_body,
    grid=(NBLK,),
    in_specs=[pl.BlockSpec((NC, 1, BLK, HALF), lambda i: (0, 0, i, 0))],
    out_specs=pl.BlockSpec((BLK, OUT_DIM), lambda i: (i, 0)),
    out_shape=jax.ShapeDtypeStruct((N_NODES, OUT_DIM), f32),
)


def _prep_edges(edge_index, etype, norm):
    gidx = etype.astype(i32) * N_NODES + edge_index[0].astype(i32)
    pad = E_PAD - E_EDGES
    gidx = jnp.concatenate([gidx, jnp.zeros((pad,), i32)])
    dst = jnp.concatenate([edge_index[1].astype(i32), jnp.zeros((pad,), i32)])
    nrm = jnp.concatenate([norm[:, 0].astype(f32), jnp.zeros((pad,), f32)])
    nrm_b = jnp.broadcast_to(nrm[:, None], (E_PAD, LANES))
    return (gidx.reshape(NS, NCHUNK, EC),
            dst.reshape(NS, NCHUNK, EC),
            nrm_b.reshape(NS, NCHUNK, EC, LANES))


def kernel(src_ids, edge_index1, etype1, norm1, edge_index2, etype2, norm2,
           emb, V1, comp1, V2, comp2):
    ids = jnp.zeros((N_PAD,), i32).at[:N_NODES].set(src_ids.astype(i32))
    ids = ids.reshape(NC * NS, XCHUNKS, XC)
    g1, d1, n1 = _prep_edges(edge_index1, etype1, norm1)
    g2, d2, n2 = _prep_edges(edge_index2, etype2, norm2)
    zeros_blk = jnp.zeros((NZROWS, HALF), f32)

    x = _gather_rows(ids, emb.astype(f32))
    y1 = _project(x[:N_NODES], V1, comp1)
    p1 = _edge_pass(y1.reshape(NC, N_REL * N_NODES, HALF), g1, d1, n1, zeros_blk)
    y2 = _combine_project(p1.reshape(NC, 1, N_ACC, HALF), V2, comp2)
    p2 = _edge_pass(y2.reshape(NC, N_REL * N_NODES, HALF), g2, d2, n2, zeros_blk)
    return _final(p2.reshape(NC, 1, N_ACC, HALF))
